# Initial kernel scaffold; baseline (speedup 1.0000x reference)
import jax
import jax.numpy as jnp
from jax.experimental import pallas as pl


def kernel(x, table):
    raise NotImplementedError("write your pallas kernel here")


# speedup vs baseline: 1.0223x; 1.0223x over previous
import functools

import jax
import jax.numpy as jnp
from jax import lax
from jax.experimental import pallas as pl
from jax.experimental.pallas import tpu as pltpu
from jax.experimental.pallas import tpu_sc as plsc

N_WORDS = 1000000
EMB_DIM = 32
BATCH = 16384
HIST = 50

NC = 2
NS = 16
NW = NC * NS

B_TOTAL = BATCH * HIST
B_PER_W = B_TOTAL // NW
CHUNK = 128
J = B_PER_W // CHUNK


def _emb_body(idx_hbm, table_hbm, out_hbm, idx_v, rows_v, sem):
    wid = lax.axis_index("s") * NC + lax.axis_index("c")
    base = wid * B_PER_W
    pltpu.sync_copy(idx_hbm.at[wid], idx_v)

    def step(g, carry):
        pltpu.async_copy(table_hbm.at[idx_v.at[g]], rows_v, sem).wait()
        pltpu.sync_copy(rows_v, out_hbm.at[pl.ds(base + g * CHUNK, CHUNK)])
        return carry

    lax.fori_loop(0, J, step, 0)


@jax.jit
def _emb_lookup(idx, table):
    mesh = plsc.VectorSubcoreMesh(core_axis_name="c", subcore_axis_name="s")
    return pl.kernel(
        _emb_body,
        out_type=jax.ShapeDtypeStruct((B_TOTAL, EMB_DIM), jnp.float32),
        mesh=mesh,
        scratch_types=[
            pltpu.VMEM((J, CHUNK), jnp.int32),
            pltpu.VMEM((CHUNK, EMB_DIM), jnp.float32),
            pltpu.SemaphoreType.DMA,
        ],
        compiler_params=pltpu.CompilerParams(use_tc_tiling_on_sc=False),
    )(idx, table)


def kernel(x, table):
    idx = x.astype(jnp.int32).reshape(NW, J, CHUNK)
    out = _emb_lookup(idx, table)
    return out.reshape(BATCH, HIST, EMB_DIM)


# speedup vs baseline: 1.1023x; 1.0783x over previous
import functools

import jax
import jax.numpy as jnp
from jax import lax
from jax.experimental import pallas as pl
from jax.experimental.pallas import tpu as pltpu
from jax.experimental.pallas import tpu_sc as plsc

N_WORDS = 1000000
EMB_DIM = 32
BATCH = 16384
HIST = 50

NC = 2
NS = 16
NW = NC * NS

B_TOTAL = BATCH * HIST
B_PER_W = B_TOTAL // NW
CHUNK = 1024
J = B_PER_W // CHUNK


def _emb_body(idx_hbm, table_hbm, out_hbm, idx_v, rows_v, sem):
    wid = lax.axis_index("s") * NC + lax.axis_index("c")
    base = wid * B_PER_W
    pltpu.sync_copy(idx_hbm.at[wid], idx_v)

    def step(g, carry):
        pltpu.async_copy(table_hbm.at[idx_v.at[g]], rows_v, sem).wait()
        pltpu.sync_copy(rows_v, out_hbm.at[pl.ds(base + g * CHUNK, CHUNK)])
        return carry

    lax.fori_loop(0, J, step, 0)


@jax.jit
def _emb_lookup(idx, table):
    mesh = plsc.VectorSubcoreMesh(core_axis_name="c", subcore_axis_name="s")
    return pl.kernel(
        _emb_body,
        out_type=jax.ShapeDtypeStruct((B_TOTAL, EMB_DIM), jnp.float32),
        mesh=mesh,
        scratch_types=[
            pltpu.VMEM((J, CHUNK), jnp.int32),
            pltpu.VMEM((CHUNK, EMB_DIM), jnp.float32),
            pltpu.SemaphoreType.DMA,
        ],
        compiler_params=pltpu.CompilerParams(use_tc_tiling_on_sc=False),
    )(idx, table)


def kernel(x, table):
    idx = x.astype(jnp.int32).reshape(NW, J, CHUNK)
    out = _emb_lookup(idx, table)
    return out.reshape(BATCH, HIST, EMB_DIM)


# speedup vs baseline: 1.1132x; 1.0098x over previous
import jax
import jax.numpy as jnp
from jax import lax
from jax.experimental import pallas as pl
from jax.experimental.pallas import tpu as pltpu
from jax.experimental.pallas import tpu_sc as plsc

N_WORDS = 1000000
EMB_DIM = 32
BATCH = 16384
HIST = 50

NC = 2
NS = 16
NW = NC * NS

B_TOTAL = BATCH * HIST
B_PER_W = B_TOTAL // NW
CHUNK = 512
G = B_PER_W // CHUNK
NBUF = 5


def _emb_body(*refs):
    idx_hbm, table_hbm, out_hbm, idx_v = refs[:4]
    bufs = refs[4:4 + NBUF]
    gsems = refs[4 + NBUF:4 + 2 * NBUF]
    ssems = refs[4 + 2 * NBUF:4 + 3 * NBUF]

    wid = lax.axis_index("s") * NC + lax.axis_index("c")
    base = wid * B_PER_W
    pltpu.sync_copy(idx_hbm.at[wid], idx_v)

    def fire_gather(g, b):
        pltpu.async_copy(table_hbm.at[idx_v.at[g]], bufs[b], gsems[b])

    def wait_gather(b):
        pltpu.make_async_copy(table_hbm.at[idx_v.at[0]], bufs[b], gsems[b]).wait()

    def fire_store(g, b):
        pltpu.async_copy(bufs[b], out_hbm.at[pl.ds(base + g * CHUNK, CHUNK)],
                         ssems[b])

    def wait_store(b):
        pltpu.make_async_copy(bufs[b], out_hbm.at[pl.ds(base, CHUNK)],
                              ssems[b]).wait()

    for b in range(NBUF):
        fire_gather(b, b)

    def group(g0, refill):
        for b in range(NBUF):
            g = g0 + b
            wait_gather(b)
            fire_store(g, b)
            if refill:
                wait_store(b)
                fire_gather(g + NBUF, b)

    def outer(i, carry):
        group(i * NBUF, True)
        return carry

    lax.fori_loop(0, G // NBUF - 1, outer, 0)
    group((G // NBUF - 1) * NBUF, False)
    for b in range(NBUF):
        wait_store(b)


@jax.jit
def _emb_lookup(idx, table):
    mesh = plsc.VectorSubcoreMesh(core_axis_name="c", subcore_axis_name="s")
    return pl.kernel(
        _emb_body,
        out_type=jax.ShapeDtypeStruct((B_TOTAL, EMB_DIM), jnp.float32),
        mesh=mesh,
        scratch_types=(
            [pltpu.VMEM((G, CHUNK), jnp.int32)]
            + [pltpu.VMEM((CHUNK, EMB_DIM), jnp.float32) for _ in range(NBUF)]
            + [pltpu.SemaphoreType.DMA for _ in range(2 * NBUF)]
        ),
        compiler_params=pltpu.CompilerParams(use_tc_tiling_on_sc=False),
    )(idx, table)


def kernel(x, table):
    idx = x.astype(jnp.int32).reshape(NW, G, CHUNK)
    out = _emb_lookup(idx, table)
    return out.reshape(BATCH, HIST, EMB_DIM)


# speedup vs baseline: 1.8103x; 1.6263x over previous
import jax
import jax.numpy as jnp
from jax import lax
from jax.experimental import pallas as pl
from jax.experimental.pallas import tpu as pltpu
from jax.experimental.pallas import tpu_sc as plsc

N_WORDS = 1000000
EMB_DIM = 32
BATCH = 16384
HIST = 50

NC = 2
NS = 16
NW = NC * NS

B_PER_W = BATCH // NW
BCHUNK = 16
CHUNK = BCHUNK * HIST
G = B_PER_W // BCHUNK
NBUF = 4


def _emb_body(*refs):
    idx_hbm, table_hbm, out_hbm, idx_v = refs[:4]
    bufs = refs[4:4 + NBUF]
    gsems = refs[4 + NBUF:4 + 2 * NBUF]
    ssems = refs[4 + 2 * NBUF:4 + 3 * NBUF]

    wid = lax.axis_index("s") * NC + lax.axis_index("c")
    b_base = wid * B_PER_W
    pltpu.sync_copy(idx_hbm.at[wid], idx_v)

    def fire_gather(g, b):
        pltpu.async_copy(table_hbm.at[idx_v.at[g]], bufs[b], gsems[b])

    def wait_gather(b):
        pltpu.make_async_copy(table_hbm.at[idx_v.at[0]], bufs[b], gsems[b]).wait()

    def fire_store(g, b):
        for k in range(BCHUNK):
            pltpu.async_copy(bufs[b].at[pl.ds(k * HIST, HIST)],
                             out_hbm.at[b_base + g * BCHUNK + k], ssems[b])

    def wait_store(b):
        for k in range(BCHUNK):
            pltpu.make_async_copy(bufs[b].at[pl.ds(k * HIST, HIST)],
                                  out_hbm.at[b_base], ssems[b]).wait()

    for b in range(NBUF):
        fire_gather(b, b)

    def group(g0, refill):
        for b in range(NBUF):
            g = g0 + b
            wait_gather(b)
            fire_store(g, b)
            if refill:
                wait_store(b)
                fire_gather(g + NBUF, b)

    def outer(i, carry):
        group(i * NBUF, True)
        return carry

    lax.fori_loop(0, G // NBUF - 1, outer, 0)
    group((G // NBUF - 1) * NBUF, False)
    for b in range(NBUF):
        wait_store(b)


@jax.jit
def _emb_lookup(idx, table):
    mesh = plsc.VectorSubcoreMesh(core_axis_name="c", subcore_axis_name="s")
    return pl.kernel(
        _emb_body,
        out_type=jax.ShapeDtypeStruct((BATCH, HIST, EMB_DIM), jnp.float32),
        mesh=mesh,
        scratch_types=(
            [pltpu.VMEM((G, CHUNK), jnp.int32)]
            + [pltpu.VMEM((CHUNK, EMB_DIM), jnp.float32) for _ in range(NBUF)]
            + [pltpu.SemaphoreType.DMA for _ in range(2 * NBUF)]
        ),
        compiler_params=pltpu.CompilerParams(use_tc_tiling_on_sc=False),
    )(idx, table)


def kernel(x, table):
    idx = x.astype(jnp.int32).reshape(NW, G, CHUNK)
    return _emb_lookup(idx, table)
